# baseline (speedup 1.0000x reference)
import functools

import jax
import jax.numpy as jnp
from jax.experimental import pallas as pl

ALPHA = 0.2
N_NODES = 10000
D_H = 512
D_OUT = 256
NH = 4


def _leaky(x):
    return jnp.where(x >= 0, x, ALPHA * x)


def _proj_kernel(nb_half, x_ref, W_ref, b_ref, asrc_ref, adst_ref,
                 h_ref, s_ref, d_ref):
    t = pl.program_id(0) // nb_half
    h = _leaky(
        jnp.dot(x_ref[...], W_ref[t], preferred_element_type=jnp.float32)
        + b_ref[t]
    )
    h_ref[...] = h
    s_ref[...] = jnp.dot(h, asrc_ref[...].T, preferred_element_type=jnp.float32)
    d_ref[...] = jnp.dot(h, adst_ref[...].T, preferred_element_type=jnp.float32)


def _edge_kernel(sg_ref, dg_ref, ex_ref):
    ex_ref[...] = jnp.exp(_leaky(sg_ref[...] + dg_ref[...]))


def _out_kernel(agg_ref, den_ref, Wout_ref, bout_ref, o_ref):
    den = den_ref[...] + 1e-16
    feat = jnp.zeros((agg_ref.shape[1], agg_ref.shape[2]), jnp.float32)
    for k in range(NH):
        a = agg_ref[k]
        a = jnp.where(jnp.isfinite(a), a, 0.0)
        feat = feat + a / den[:, k:k + 1]
    feat = feat / NH
    out = jnp.dot(feat, Wout_ref[...], preferred_element_type=jnp.float32)
    out = out + bout_ref[0]
    nrm = jnp.sqrt(jnp.sum(out * out, axis=1, keepdims=True))
    o_ref[...] = out / jnp.maximum(nrm, 1e-12)


def kernel(x0, x1, W0, b0, W1, b1, a_src, a_dst, Wout, bout, edge_index):
    x = jnp.concatenate([x0, x1], axis=0)
    Ws = jnp.stack([W0, W1])
    bs = jnp.stack([b0, b1])
    n, d_in = x.shape
    BLK = 1000
    NB = n // BLK

    h, s, d = pl.pallas_call(
        functools.partial(_proj_kernel, NB // 2),
        grid=(NB,),
        in_specs=[
            pl.BlockSpec((BLK, d_in), lambda i: (i, 0)),
            pl.BlockSpec((2, d_in, D_H), lambda i: (0, 0, 0)),
            pl.BlockSpec((2, D_H), lambda i: (0, 0)),
            pl.BlockSpec((NH, D_H), lambda i: (0, 0)),
            pl.BlockSpec((NH, D_H), lambda i: (0, 0)),
        ],
        out_specs=[
            pl.BlockSpec((BLK, D_H), lambda i: (i, 0)),
            pl.BlockSpec((BLK, NH), lambda i: (i, 0)),
            pl.BlockSpec((BLK, NH), lambda i: (i, 0)),
        ],
        out_shape=[
            jax.ShapeDtypeStruct((n, D_H), jnp.float32),
            jax.ShapeDtypeStruct((n, NH), jnp.float32),
            jax.ShapeDtypeStruct((n, NH), jnp.float32),
        ],
    )(x, Ws, bs, a_src, a_dst)

    src, dst = edge_index[0], edge_index[1]
    sg = jnp.take(s, src, axis=0)
    dg = jnp.take(d, dst, axis=0)
    n_edges = sg.shape[0]
    EBLK = 20000
    ex = pl.pallas_call(
        _edge_kernel,
        grid=(n_edges // EBLK,),
        in_specs=[
            pl.BlockSpec((EBLK, NH), lambda i: (i, 0)),
            pl.BlockSpec((EBLK, NH), lambda i: (i, 0)),
        ],
        out_specs=pl.BlockSpec((EBLK, NH), lambda i: (i, 0)),
        out_shape=jax.ShapeDtypeStruct((n_edges, NH), jnp.float32),
    )(sg, dg)

    denom = jax.ops.segment_sum(ex, dst, num_segments=N_NODES)
    hs = jnp.take(h, src, axis=0)
    aggs = []
    for k in range(NH):
        aggs.append(jax.ops.segment_max(ex[:, k:k + 1] * hs, dst,
                                        num_segments=N_NODES))
    agg = jnp.stack(aggs, axis=0)

    out = pl.pallas_call(
        _out_kernel,
        grid=(NB,),
        in_specs=[
            pl.BlockSpec((NH, BLK, D_H), lambda i: (0, i, 0)),
            pl.BlockSpec((BLK, NH), lambda i: (i, 0)),
            pl.BlockSpec((D_H, D_OUT), lambda i: (0, 0)),
            pl.BlockSpec((1, D_OUT), lambda i: (0, 0)),
        ],
        out_specs=pl.BlockSpec((BLK, D_OUT), lambda i: (i, 0)),
        out_shape=jax.ShapeDtypeStruct((n, D_OUT), jnp.float32),
    )(agg, denom, Wout, bout.reshape(1, D_OUT))
    return out


# speedup vs baseline: 1.0455x; 1.0455x over previous
import functools

import jax
import jax.numpy as jnp
from jax.experimental import pallas as pl

ALPHA = 0.2
N_NODES = 10000
D_H = 512
D_OUT = 256
NH = 4


def _leaky(x):
    return jnp.where(x >= 0, x, ALPHA * x)


def _proj_kernel(nb_half, x_ref, W_ref, b_ref, asrc_ref, adst_ref,
                 h_ref, s_ref, d_ref):
    t = pl.program_id(0) // nb_half
    h = _leaky(
        jnp.dot(x_ref[...], W_ref[t], preferred_element_type=jnp.float32)
        + b_ref[t]
    )
    h_ref[...] = h
    s_ref[...] = jnp.dot(h, asrc_ref[...].T, preferred_element_type=jnp.float32)
    d_ref[...] = jnp.dot(h, adst_ref[...].T, preferred_element_type=jnp.float32)


def _edge_kernel(sg_ref, dg_ref, ex_ref):
    ex_ref[...] = jnp.exp(_leaky(sg_ref[...] + dg_ref[...]))


def _out_kernel(a0_ref, a1_ref, a2_ref, a3_ref, den_ref, Wout_ref, bout_ref,
                o_ref):
    den = den_ref[...] + 1e-16
    feat = jnp.zeros(a0_ref.shape, jnp.float32)
    for k, a_ref in enumerate((a0_ref, a1_ref, a2_ref, a3_ref)):
        a = a_ref[...]
        a = jnp.where(jnp.isfinite(a), a, 0.0)
        feat = feat + a / den[:, k:k + 1]
    feat = feat / NH
    out = jnp.dot(feat, Wout_ref[...], preferred_element_type=jnp.float32)
    out = out + bout_ref[0]
    nrm = jnp.sqrt(jnp.sum(out * out, axis=1, keepdims=True))
    o_ref[...] = out / jnp.maximum(nrm, 1e-12)


def kernel(x0, x1, W0, b0, W1, b1, a_src, a_dst, Wout, bout, edge_index):
    x = jnp.concatenate([x0, x1], axis=0)
    Ws = jnp.stack([W0, W1])
    bs = jnp.stack([b0, b1])
    n, d_in = x.shape
    BLK = 1000
    NB = n // BLK

    h, s, d = pl.pallas_call(
        functools.partial(_proj_kernel, NB // 2),
        grid=(NB,),
        in_specs=[
            pl.BlockSpec((BLK, d_in), lambda i: (i, 0)),
            pl.BlockSpec((2, d_in, D_H), lambda i: (0, 0, 0)),
            pl.BlockSpec((2, D_H), lambda i: (0, 0)),
            pl.BlockSpec((NH, D_H), lambda i: (0, 0)),
            pl.BlockSpec((NH, D_H), lambda i: (0, 0)),
        ],
        out_specs=[
            pl.BlockSpec((BLK, D_H), lambda i: (i, 0)),
            pl.BlockSpec((BLK, NH), lambda i: (i, 0)),
            pl.BlockSpec((BLK, NH), lambda i: (i, 0)),
        ],
        out_shape=[
            jax.ShapeDtypeStruct((n, D_H), jnp.float32),
            jax.ShapeDtypeStruct((n, NH), jnp.float32),
            jax.ShapeDtypeStruct((n, NH), jnp.float32),
        ],
    )(x, Ws, bs, a_src, a_dst)

    src, dst = edge_index[0], edge_index[1]
    sg = jnp.take(s, src, axis=0)
    dg = jnp.take(d, dst, axis=0)
    n_edges = sg.shape[0]
    EBLK = 20000
    ex = pl.pallas_call(
        _edge_kernel,
        grid=(n_edges // EBLK,),
        in_specs=[
            pl.BlockSpec((EBLK, NH), lambda i: (i, 0)),
            pl.BlockSpec((EBLK, NH), lambda i: (i, 0)),
        ],
        out_specs=pl.BlockSpec((EBLK, NH), lambda i: (i, 0)),
        out_shape=jax.ShapeDtypeStruct((n_edges, NH), jnp.float32),
    )(sg, dg)

    denom = jax.ops.segment_sum(ex, dst, num_segments=N_NODES)
    hs = jnp.take(h, src, axis=0)
    aggs = []
    for k in range(NH):
        aggs.append(jax.ops.segment_max(ex[:, k:k + 1] * hs, dst,
                                        num_segments=N_NODES))

    out = pl.pallas_call(
        _out_kernel,
        grid=(NB,),
        in_specs=[
            pl.BlockSpec((BLK, D_H), lambda i: (i, 0)),
            pl.BlockSpec((BLK, D_H), lambda i: (i, 0)),
            pl.BlockSpec((BLK, D_H), lambda i: (i, 0)),
            pl.BlockSpec((BLK, D_H), lambda i: (i, 0)),
            pl.BlockSpec((BLK, NH), lambda i: (i, 0)),
            pl.BlockSpec((D_H, D_OUT), lambda i: (0, 0)),
            pl.BlockSpec((1, D_OUT), lambda i: (0, 0)),
        ],
        out_specs=pl.BlockSpec((BLK, D_OUT), lambda i: (i, 0)),
        out_shape=jax.ShapeDtypeStruct((n, D_OUT), jnp.float32),
    )(aggs[0], aggs[1], aggs[2], aggs[3], denom, Wout, bout.reshape(1, D_OUT))
    return out
